# baseline (speedup 1.0000x reference)
import functools

import jax
import jax.numpy as jnp
from jax import lax
from jax.experimental import pallas as pl
from jax.experimental.pallas import tpu as pltpu
from jax.experimental.pallas import tpu_sc as plsc

N_FIELDS = 26
VOCAB = 100000
EMB_DIM = 32

NC = 2
NS = 16
NW = NC * NS

LINE = 128
RPL = LINE // EMB_DIM
CHUNK = 128
NBUF = 4


@functools.lru_cache(maxsize=None)
def _make_gather(n_rows: int):
    assert n_rows % (NW * CHUNK) == 0
    n_chunks = n_rows // (NW * CHUNK)
    assert n_chunks % NBUF == 0
    lines_per_chunk = CHUNK // RPL

    mesh = plsc.VectorSubcoreMesh(
        core_axis_name="c", subcore_axis_name="s", num_cores=NC, num_subcores=NS
    )

    def body(table, idx, out, idx_v, lidx_v, bufs, stage, gsem, ssem):
        wid = lax.axis_index("s") * NC + lax.axis_index("c")
        out_base = wid * n_chunks * lines_per_chunk
        pltpu.sync_copy(idx.at[wid], idx_v)

        @pl.loop(0, n_chunks)
        def _(j):
            for k in range(CHUNK // 16):
                sl = pl.ds(k * 16, 16)
                lidx_v[j, sl] = jax.lax.shift_right_logical(idx_v[j, sl], 2)

        def start_gather(j, b):
            pltpu.async_copy(table.at[lidx_v.at[j]], bufs.at[b], gsem.at[b])

        def wait_gather(b):
            pltpu.make_async_copy(
                table.at[pl.ds(0, CHUNK)], bufs.at[b], gsem.at[b]
            ).wait()

        def start_store(j, b):
            pltpu.async_copy(
                stage.at[b],
                out.at[pl.ds(out_base + j * lines_per_chunk, lines_per_chunk)],
                ssem.at[b],
            )

        def wait_store(b):
            pltpu.make_async_copy(
                stage.at[b], out.at[pl.ds(0, lines_per_chunk)], ssem.at[b]
            ).wait()

        def extract(j, b):
            lanes = lax.iota(jnp.int32, 16)
            rloc = lax.shift_right_logical(lanes, 2)
            dcol0 = lax.shift_left(lanes & (RPL - 1), 5)

            @pl.loop(0, CHUNK // 16)
            def _(k):
                gvec = idx_v[j, pl.ds(k * 16, 16)]
                src_col = lax.shift_left(gvec & (RPL - 1), 5)
                buf_win = bufs.at[b].at[pl.ds(k * 16, 16)]
                st_win = stage.at[b].at[pl.ds(k * RPL, RPL)]
                for e in range(EMB_DIM):
                    vals = plsc.load_gather(buf_win, [lanes, src_col + e])
                    plsc.store_scatter(st_win, [rloc, dcol0 + e], vals)

        for b in range(NBUF):
            start_gather(b, b)

        @pl.loop(0, n_chunks, step=NBUF)
        def _(g0):
            for b in range(NBUF):
                j = g0 + b
                wait_gather(b)
                extract(j, b)
                start_store(j, b)
                nj = j + NBUF

                @pl.when(nj < n_chunks)
                def _():
                    wait_store(b)
                    start_gather(nj, b)

        for b in range(NBUF):
            wait_store(b)

    return pl.kernel(
        body,
        out_type=jax.ShapeDtypeStruct((n_rows // RPL, LINE), jnp.float32),
        mesh=mesh,
        compiler_params=pltpu.CompilerParams(
            use_tc_tiling_on_sc=True, needs_layout_passes=False
        ),
        scratch_types=[
            pltpu.VMEM((n_chunks, CHUNK), jnp.int32),
            pltpu.VMEM((n_chunks, CHUNK), jnp.int32),
            pltpu.VMEM((NBUF, CHUNK, LINE), jnp.float32),
            pltpu.VMEM((NBUF, lines_per_chunk, LINE), jnp.float32),
            pltpu.SemaphoreType.DMA((NBUF,)),
            pltpu.SemaphoreType.DMA((NBUF,)),
        ],
    )


def kernel(cat_features, tables):
    batch = cat_features.shape[0]
    n_rows = batch * N_FIELDS
    cat = cat_features.astype(jnp.int32)
    offs = jnp.arange(N_FIELDS, dtype=jnp.int32) * VOCAB
    idx = (cat + offs[None, :]).reshape(NW, n_rows // (NW * CHUNK), CHUNK)
    t_lines = tables.reshape(N_FIELDS * VOCAB // RPL, LINE)
    out = _make_gather(n_rows)(t_lines, idx)
    return out.reshape(batch, N_FIELDS * EMB_DIM)


# speedup vs baseline: 3.3114x; 3.3114x over previous
import functools

import jax
import jax.numpy as jnp
from jax import lax
from jax.experimental import pallas as pl
from jax.experimental.pallas import tpu as pltpu
from jax.experimental.pallas import tpu_sc as plsc

N_FIELDS = 26
VOCAB = 100000
EMB_DIM = 32

NC = 2
NS = 16
NW = NC * NS

BCHUNK = 2048


@functools.lru_cache(maxsize=None)
def _make_lookup(batch: int):
    n_rows = N_FIELDS * EMB_DIM
    units = n_rows // NW
    assert units * NW == n_rows and batch % BCHUNK == 0
    n_bchunks = batch // BCHUNK

    mesh = plsc.VectorSubcoreMesh(
        core_axis_name="c", subcore_axis_name="s", num_cores=NC, num_subcores=NS
    )

    def body(tt, catT, out, vec, idxb, outb):
        wid = lax.axis_index("s") * NC + lax.axis_index("c")

        @pl.loop(0, units)
        def _(t):
            u = wid * units + t
            f = u // EMB_DIM
            e = lax.rem(u, EMB_DIM)
            pltpu.sync_copy(tt.at[f, e], vec)

            @pl.loop(0, n_bchunks)
            def _(c):
                pltpu.sync_copy(catT.at[f, pl.ds(c * BCHUNK, BCHUNK)], idxb)

                @pl.loop(0, BCHUNK // 16, unroll=8)
                def _(k):
                    iv = idxb[pl.ds(k * 16, 16)]
                    outb[pl.ds(k * 16, 16)] = plsc.load_gather(vec, [iv])

                pltpu.sync_copy(outb, out.at[u, pl.ds(c * BCHUNK, BCHUNK)])

    return pl.kernel(
        body,
        out_type=jax.ShapeDtypeStruct((n_rows, batch), jnp.float32),
        mesh=mesh,
        compiler_params=pltpu.CompilerParams(
            use_tc_tiling_on_sc=True, needs_layout_passes=False
        ),
        scratch_types=[
            pltpu.VMEM((VOCAB,), jnp.float32),
            pltpu.VMEM((BCHUNK,), jnp.int32),
            pltpu.VMEM((BCHUNK,), jnp.float32),
        ],
    )


def kernel(cat_features, tables):
    batch = cat_features.shape[0]
    cat = cat_features.astype(jnp.int32)
    tt = jnp.transpose(tables, (0, 2, 1))
    catT = jnp.transpose(cat, (1, 0))
    out = _make_lookup(batch)(tt, catT)
    return jnp.transpose(out, (1, 0))


# speedup vs baseline: 4.7096x; 1.4222x over previous
import functools

import jax
import jax.numpy as jnp
from jax import lax
from jax.experimental import pallas as pl
from jax.experimental.pallas import tpu as pltpu
from jax.experimental.pallas import tpu_sc as plsc

N_FIELDS = 26
VOCAB = 100000
EMB_DIM = 32

NC = 2
NS = 16
NW = NC * NS

BCHUNK = 2048


@functools.lru_cache(maxsize=None)
def _make_lookup(batch: int):
    n_rows = N_FIELDS * EMB_DIM
    units = n_rows // NW
    assert units * NW == n_rows and batch % (2 * BCHUNK) == 0
    n_bchunks = batch // BCHUNK

    mesh = plsc.VectorSubcoreMesh(
        core_axis_name="c", subcore_axis_name="s", num_cores=NC, num_subcores=NS
    )

    def body(tt, catT, out, vec, idxb0, idxb1, outb0, outb1, vsem, isem, osem):
        wid = lax.axis_index("s") * NC + lax.axis_index("c")
        idxbs = (idxb0, idxb1)
        outbs = (outb0, outb1)

        def start_idx(f, c, b):
            pltpu.async_copy(
                catT.at[f, pl.ds(c * BCHUNK, BCHUNK)], idxbs[b], isem.at[b]
            )

        def wait_idx(b):
            pltpu.make_async_copy(
                catT.at[0, pl.ds(0, BCHUNK)], idxbs[b], isem.at[b]
            ).wait()

        def start_out(u, c, b):
            pltpu.async_copy(
                outbs[b], out.at[u, pl.ds(c * BCHUNK, BCHUNK)], osem.at[b]
            )

        def wait_out(b):
            pltpu.make_async_copy(
                outbs[b], out.at[0, pl.ds(0, BCHUNK)], osem.at[b]
            ).wait()

        def gather_chunk(b):
            ib = idxbs[b]
            ob = outbs[b]

            @pl.loop(0, BCHUNK // 16, unroll=8)
            def _(k):
                iv = ib[pl.ds(k * 16, 16)]
                ob[pl.ds(k * 16, 16)] = plsc.load_gather(vec, [iv])

        @pl.loop(0, units)
        def _(t):
            u = wid * units + t
            f = u // EMB_DIM
            e = lax.rem(u, EMB_DIM)
            pltpu.async_copy(tt.at[f, e], vec, vsem)
            start_idx(f, 0, 0)
            start_idx(f, 1, 1)
            pltpu.make_async_copy(tt.at[0, 0], vec, vsem).wait()

            @pl.loop(0, n_bchunks, step=2)
            def _(c0):
                for b in range(2):
                    c = c0 + b
                    wait_idx(b)

                    @pl.when(t * n_bchunks + c > 1)
                    def _():
                        wait_out(b)

                    gather_chunk(b)
                    start_out(u, c, b)
                    nc = c + 2

                    @pl.when(nc < n_bchunks)
                    def _():
                        start_idx(f, nc, b)

        for b in range(2):
            wait_out(b)

    return pl.kernel(
        body,
        out_type=jax.ShapeDtypeStruct((n_rows, batch), jnp.float32),
        mesh=mesh,
        compiler_params=pltpu.CompilerParams(
            use_tc_tiling_on_sc=True, needs_layout_passes=False
        ),
        scratch_types=[
            pltpu.VMEM((VOCAB,), jnp.float32),
            pltpu.VMEM((BCHUNK,), jnp.int32),
            pltpu.VMEM((BCHUNK,), jnp.int32),
            pltpu.VMEM((BCHUNK,), jnp.float32),
            pltpu.VMEM((BCHUNK,), jnp.float32),
            pltpu.SemaphoreType.DMA,
            pltpu.SemaphoreType.DMA((2,)),
            pltpu.SemaphoreType.DMA((2,)),
        ],
    )


def kernel(cat_features, tables):
    batch = cat_features.shape[0]
    cat = cat_features.astype(jnp.int32)
    tt = jnp.transpose(tables, (0, 2, 1))
    catT = jnp.transpose(cat, (1, 0))
    out = _make_lookup(batch)(tt, catT)
    return jnp.transpose(out, (1, 0))
